# Initial kernel scaffold; baseline (speedup 1.0000x reference)
import jax
import jax.numpy as jnp
from jax.experimental import pallas as pl


def kernel(graph, features, W1, b1, W2, b2):
    raise NotImplementedError("write your pallas kernel here")


# speedup vs baseline: 9.7481x; 9.7481x over previous
import functools

import jax
import jax.numpy as jnp
from jax import lax
from jax.experimental import pallas as pl
from jax.experimental.pallas import tpu as pltpu
from jax.experimental.pallas import tpu_sc as plsc

ALPHA = 1.0
LAMBD = 1.0
K_ITERS = 8

NW = 32
CHUNK = 128


def _mesh():
    return plsc.VectorSubcoreMesh(core_axis_name="c", subcore_axis_name="s",
                                  num_cores=2, num_subcores=16)


def _make_sc_propagate(n_pad, d, n_chunks):
    rows_per_sub = n_pad // 16

    @functools.partial(
        pl.kernel,
        mesh=_mesh(),
        compiler_params=pltpu.CompilerParams(use_tc_tiling_on_sc=False),
        out_type=jax.ShapeDtypeStruct((2, n_pad, d), jnp.float32),
        scratch_types=[
            pltpu.VMEM((n_chunks, CHUNK), jnp.int32),
            pltpu.VMEM((n_chunks, CHUNK), jnp.int32),
            pltpu.VMEM((CHUNK, d), jnp.float32),
            pltpu.VMEM((CHUNK, d), jnp.float32),
            pltpu.VMEM_SHARED((n_pad, d), jnp.float32),
            pltpu.SemaphoreType.DMA,
            pltpu.SemaphoreType.DMA,
        ],
    )
    def prop(g_hbm, src_hbm, dst_hbm, zeros_hbm, out_hbm,
             src_v, dst_v, buf_a, buf_b, acc_sh, sem_a, sem_b):
        c = lax.axis_index("c")
        s = lax.axis_index("s")
        wid = s * 2 + c

        pltpu.sync_copy(src_hbm.at[wid], src_v)
        pltpu.sync_copy(dst_hbm.at[wid], dst_v)
        lo = s * rows_per_sub
        pltpu.sync_copy(zeros_hbm.at[pl.ds(lo, rows_per_sub)],
                        acc_sh.at[pl.ds(lo, rows_per_sub)])
        plsc.subcore_barrier()

        cp_a = pltpu.async_copy(g_hbm.at[src_v.at[0]], buf_a, sem_a)

        def body(t, _):
            j0 = 2 * t
            pltpu.make_async_copy(g_hbm.at[src_v.at[j0]], buf_a, sem_a).wait()
            pltpu.async_copy(g_hbm.at[src_v.at[j0 + 1]], buf_b, sem_b)
            pltpu.sync_copy(buf_a, acc_sh.at[dst_v.at[j0]], add=True)
            pltpu.make_async_copy(g_hbm.at[src_v.at[j0 + 1]], buf_b, sem_b).wait()
            pltpu.async_copy(g_hbm.at[src_v.at[j0 + 2]], buf_a, sem_a)
            pltpu.sync_copy(buf_b, acc_sh.at[dst_v.at[j0 + 1]], add=True)
            return _

        lax.fori_loop(0, n_chunks // 2 - 1, body, 0)
        j0 = n_chunks - 2
        pltpu.make_async_copy(g_hbm.at[src_v.at[j0]], buf_a, sem_a).wait()
        cp_b = pltpu.async_copy(g_hbm.at[src_v.at[j0 + 1]], buf_b, sem_b)
        pltpu.sync_copy(buf_a, acc_sh.at[dst_v.at[j0]], add=True)
        cp_b.wait()
        pltpu.sync_copy(buf_b, acc_sh.at[dst_v.at[j0 + 1]], add=True)
        del cp_a

        plsc.subcore_barrier()
        pltpu.sync_copy(acc_sh.at[pl.ds(lo, rows_per_sub)],
                        out_hbm.at[c, pl.ds(lo, rows_per_sub)])

    return prop


def _dinv_body(deg_ref, dinv_ref, dinv2_ref):
    dsum = deg_ref[0] + deg_ref[1]
    di = lax.rsqrt(jnp.maximum(dsum[:, 0:1], 1.0))
    dinv_ref[...] = di
    dinv2_ref[...] = di * di


def _dinv_call(deg_p, n):
    blk = n // 10
    return pl.pallas_call(
        _dinv_body,
        grid=(10,),
        in_specs=[pl.BlockSpec((2, blk, 64), lambda i: (0, i, 0))],
        out_specs=[pl.BlockSpec((blk, 1), lambda i: (i, 0)),
                   pl.BlockSpec((blk, 1), lambda i: (i, 0))],
        out_shape=[jax.ShapeDtypeStruct((n, 1), jnp.float32),
                   jax.ShapeDtypeStruct((n, 1), jnp.float32)],
    )(deg_p)


def _mlp_body(x_ref, w_ref, b_ref, dinv_ref, h_ref, hs_ref):
    acc = jnp.dot(x_ref[...], w_ref[...], preferred_element_type=jnp.float32)
    h = jnp.maximum(acc + b_ref[...], 0.0)
    h_ref[...] = h
    hs_ref[...] = h * dinv_ref[...]


def _mlp_call(x, w, b, dinv):
    n, f_in = x.shape
    f_out = w.shape[1]
    blk = n // 10
    return pl.pallas_call(
        _mlp_body,
        grid=(10,),
        in_specs=[pl.BlockSpec((blk, f_in), lambda i: (i, 0)),
                  pl.BlockSpec((f_in, f_out), lambda i: (0, 0)),
                  pl.BlockSpec((1, f_out), lambda i: (0, 0)),
                  pl.BlockSpec((blk, 1), lambda i: (i, 0))],
        out_specs=[pl.BlockSpec((blk, f_out), lambda i: (i, 0)),
                   pl.BlockSpec((blk, f_out), lambda i: (i, 0))],
        out_shape=[jax.ShapeDtypeStruct((n, f_out), jnp.float32),
                   jax.ShapeDtypeStruct((n, f_out), jnp.float32)],
    )(x, w, b.reshape(1, f_out), dinv)


def _combine_body(acc_ref, h0_ref, h0s_ref, dinv_ref, dinv2_ref, g_ref, h_ref):
    ssum = acc_ref[0] + acc_ref[1]
    g_ref[...] = 0.5 * (h0s_ref[...] + dinv2_ref[...] * ssum)
    h_ref[...] = 0.5 * (h0_ref[...] + dinv_ref[...] * ssum)


def _combine_call(acc_p, h0, h0s, dinv, dinv2):
    n, d = h0.shape
    blk = n // 10
    return pl.pallas_call(
        _combine_body,
        grid=(10,),
        in_specs=[pl.BlockSpec((2, blk, d), lambda i: (0, i, 0)),
                  pl.BlockSpec((blk, d), lambda i: (i, 0)),
                  pl.BlockSpec((blk, d), lambda i: (i, 0)),
                  pl.BlockSpec((blk, 1), lambda i: (i, 0)),
                  pl.BlockSpec((blk, 1), lambda i: (i, 0))],
        out_specs=[pl.BlockSpec((blk, d), lambda i: (i, 0)),
                   pl.BlockSpec((blk, d), lambda i: (i, 0))],
        out_shape=[jax.ShapeDtypeStruct((n, d), jnp.float32),
                   jax.ShapeDtypeStruct((n, d), jnp.float32)],
    )(acc_p, h0, h0s, dinv, dinv2)


def kernel(graph, features, W1, b1, W2, b2):
    n, _ = features.shape
    e = graph.shape[1]
    d = W1.shape[1]

    e_tot = e + n
    n_chunks = -(-e_tot // (NW * CHUNK))
    if n_chunks % 2:
        n_chunks += 1
    e_pad = NW * n_chunks * CHUNK
    loop_idx = jnp.arange(n, dtype=jnp.int32)
    pad_src = jnp.zeros((e_pad - e_tot,), dtype=jnp.int32)
    pad_dst = jnp.full((e_pad - e_tot,), n, dtype=jnp.int32)
    src_l = jnp.concatenate([graph[0], loop_idx, pad_src]).reshape(NW, n_chunks, CHUNK)
    dst_l = jnp.concatenate([graph[1], loop_idx, pad_dst]).reshape(NW, n_chunks, CHUNK)

    n_pad = 128 * (-(-(n + 1) // 128))
    zeros_d = jnp.zeros((n_pad, d), dtype=jnp.float32)
    ones_nd = jnp.ones((n, d), dtype=jnp.float32)

    sc_prop = _make_sc_propagate(n_pad, d, n_chunks)

    deg_p = sc_prop(ones_nd, src_l, dst_l, zeros_d)
    dinv, dinv2 = _dinv_call(deg_p, n)

    h0, h0s = _mlp_call(features, W1, b1, dinv)
    g = h0s
    for _ in range(K_ITERS):
        acc_p = sc_prop(g, src_l, dst_l, zeros_d)
        g, h = _combine_call(acc_p, h0, h0s, dinv, dinv2)
    h2, h2s = _mlp_call(h, W2, b2, dinv)
    g = h2s
    for _ in range(K_ITERS):
        acc_p = sc_prop(g, src_l, dst_l, zeros_d)
        g, h = _combine_call(acc_p, h2, h2s, dinv, dinv2)
    return h


# speedup vs baseline: 11.0599x; 1.1346x over previous
import functools

import jax
import jax.numpy as jnp
from jax import lax
from jax.experimental import pallas as pl
from jax.experimental.pallas import tpu as pltpu
from jax.experimental.pallas import tpu_sc as plsc

ALPHA = 1.0
LAMBD = 1.0
K_ITERS = 8

NW = 32
CHUNK = 128


def _mesh():
    return plsc.VectorSubcoreMesh(core_axis_name="c", subcore_axis_name="s",
                                  num_cores=2, num_subcores=16)


def _make_sc_propagate(n_pad, d, n_chunks):
    rows_per_sub = n_pad // 16

    @functools.partial(
        pl.kernel,
        mesh=_mesh(),
        compiler_params=pltpu.CompilerParams(use_tc_tiling_on_sc=False),
        out_type=jax.ShapeDtypeStruct((2, n_pad, d), jnp.float32),
        scratch_types=[
            pltpu.VMEM((n_chunks, CHUNK), jnp.int32),
            pltpu.VMEM((n_chunks, CHUNK), jnp.int32),
            [pltpu.VMEM((CHUNK, d), jnp.float32) for _ in range(6)],
            pltpu.VMEM_SHARED((n_pad, d), jnp.float32),
            pltpu.SemaphoreType.DMA,
            pltpu.SemaphoreType.DMA,
        ],
    )
    def prop(g_hbm, src_hbm, dst_hbm, zeros_hbm, out_hbm,
             src_v, dst_v, bufs, acc_sh, sem_g, sem_s):
        c = lax.axis_index("c")
        s = lax.axis_index("s")
        wid = s * 2 + c

        pltpu.sync_copy(src_hbm.at[wid], src_v)
        pltpu.sync_copy(dst_hbm.at[wid], dst_v)
        lo = s * rows_per_sub
        pltpu.sync_copy(zeros_hbm.at[pl.ds(lo, rows_per_sub)],
                        acc_sh.at[pl.ds(lo, rows_per_sub)])
        plsc.subcore_barrier()

        def fire_g(j, buf):
            pltpu.async_copy(g_hbm.at[src_v.at[j]], buf, sem_g)

        def wait_g(buf):
            pltpu.make_async_copy(g_hbm.at[src_v.at[0]], buf, sem_g).wait()

        def fire_s(j, buf):
            pltpu.async_copy(buf, acc_sh.at[dst_v.at[j]], sem_s, add=True)

        def wait_s(buf):
            pltpu.make_async_copy(buf, acc_sh.at[dst_v.at[0]], sem_s).wait()

        def half(base, grp, other, other_base):
            for i in range(3):
                fire_g(other_base + i, other[i])
            for i in range(3):
                wait_g(grp[i])
            for i in range(3):
                fire_s(base + i, grp[i])
            for i in range(3):
                wait_s(grp[i])

        x_grp, y_grp = bufs[0:3], bufs[3:6]
        for i in range(3):
            fire_g(i, x_grp[i])

        def body(u, _):
            b0 = 6 * u
            half(b0, x_grp, y_grp, b0 + 3)
            half(b0 + 3, y_grp, x_grp, b0 + 6)
            return _

        lax.fori_loop(0, n_chunks // 6 - 1, body, 0)
        b0 = n_chunks - 6
        half(b0, x_grp, y_grp, b0 + 3)
        for i in range(3):
            wait_g(y_grp[i])
        for i in range(3):
            fire_s(b0 + 3 + i, y_grp[i])
        for i in range(3):
            wait_s(y_grp[i])

        plsc.subcore_barrier()
        pltpu.sync_copy(acc_sh.at[pl.ds(lo, rows_per_sub)],
                        out_hbm.at[c, pl.ds(lo, rows_per_sub)])

    return prop


def _dinv_body(deg_ref, dinv_ref, dinv2_ref):
    dsum = deg_ref[0] + deg_ref[1]
    di = lax.rsqrt(jnp.maximum(dsum[:, 0:1], 1.0))
    dinv_ref[...] = di
    dinv2_ref[...] = di * di


def _dinv_call(deg_p, n):
    blk = n // 10
    return pl.pallas_call(
        _dinv_body,
        grid=(10,),
        in_specs=[pl.BlockSpec((2, blk, 64), lambda i: (0, i, 0))],
        out_specs=[pl.BlockSpec((blk, 1), lambda i: (i, 0)),
                   pl.BlockSpec((blk, 1), lambda i: (i, 0))],
        out_shape=[jax.ShapeDtypeStruct((n, 1), jnp.float32),
                   jax.ShapeDtypeStruct((n, 1), jnp.float32)],
    )(deg_p)


def _mlp_body(x_ref, w_ref, b_ref, dinv_ref, h_ref, hs_ref):
    acc = jnp.dot(x_ref[...], w_ref[...], preferred_element_type=jnp.float32)
    h = jnp.maximum(acc + b_ref[...], 0.0)
    h_ref[...] = h
    hs_ref[...] = h * dinv_ref[...]


def _mlp_call(x, w, b, dinv):
    n, f_in = x.shape
    f_out = w.shape[1]
    blk = n // 10
    return pl.pallas_call(
        _mlp_body,
        grid=(10,),
        in_specs=[pl.BlockSpec((blk, f_in), lambda i: (i, 0)),
                  pl.BlockSpec((f_in, f_out), lambda i: (0, 0)),
                  pl.BlockSpec((1, f_out), lambda i: (0, 0)),
                  pl.BlockSpec((blk, 1), lambda i: (i, 0))],
        out_specs=[pl.BlockSpec((blk, f_out), lambda i: (i, 0)),
                   pl.BlockSpec((blk, f_out), lambda i: (i, 0))],
        out_shape=[jax.ShapeDtypeStruct((n, f_out), jnp.float32),
                   jax.ShapeDtypeStruct((n, f_out), jnp.float32)],
    )(x, w, b.reshape(1, f_out), dinv)


def _combine_body(acc_ref, h0_ref, h0s_ref, dinv_ref, dinv2_ref, g_ref, h_ref):
    ssum = acc_ref[0] + acc_ref[1]
    g_ref[...] = 0.5 * (h0s_ref[...] + dinv2_ref[...] * ssum)
    h_ref[...] = 0.5 * (h0_ref[...] + dinv_ref[...] * ssum)


def _combine_call(acc_p, h0, h0s, dinv, dinv2):
    n, d = h0.shape
    blk = n // 10
    return pl.pallas_call(
        _combine_body,
        grid=(10,),
        in_specs=[pl.BlockSpec((2, blk, d), lambda i: (0, i, 0)),
                  pl.BlockSpec((blk, d), lambda i: (i, 0)),
                  pl.BlockSpec((blk, d), lambda i: (i, 0)),
                  pl.BlockSpec((blk, 1), lambda i: (i, 0)),
                  pl.BlockSpec((blk, 1), lambda i: (i, 0))],
        out_specs=[pl.BlockSpec((blk, d), lambda i: (i, 0)),
                   pl.BlockSpec((blk, d), lambda i: (i, 0))],
        out_shape=[jax.ShapeDtypeStruct((n, d), jnp.float32),
                   jax.ShapeDtypeStruct((n, d), jnp.float32)],
    )(acc_p, h0, h0s, dinv, dinv2)


def kernel(graph, features, W1, b1, W2, b2):
    n, _ = features.shape
    e = graph.shape[1]
    d = W1.shape[1]

    e_tot = e + n
    n_chunks = -(-e_tot // (NW * CHUNK))
    n_chunks = 6 * (-(-n_chunks // 6))
    e_pad = NW * n_chunks * CHUNK
    loop_idx = jnp.arange(n, dtype=jnp.int32)
    pad_src = jnp.zeros((e_pad - e_tot,), dtype=jnp.int32)
    pad_dst = jnp.full((e_pad - e_tot,), n, dtype=jnp.int32)
    src_l = jnp.concatenate([graph[0], loop_idx, pad_src]).reshape(NW, n_chunks, CHUNK)
    dst_l = jnp.concatenate([graph[1], loop_idx, pad_dst]).reshape(NW, n_chunks, CHUNK)

    n_pad = 128 * (-(-(n + 1) // 128))
    zeros_d = jnp.zeros((n_pad, d), dtype=jnp.float32)
    ones_nd = jnp.ones((n, d), dtype=jnp.float32)

    sc_prop = _make_sc_propagate(n_pad, d, n_chunks)

    deg_p = sc_prop(ones_nd, src_l, dst_l, zeros_d)
    dinv, dinv2 = _dinv_call(deg_p, n)

    h0, h0s = _mlp_call(features, W1, b1, dinv)
    g = h0s
    for _ in range(K_ITERS):
        acc_p = sc_prop(g, src_l, dst_l, zeros_d)
        g, h = _combine_call(acc_p, h0, h0s, dinv, dinv2)
    h2, h2s = _mlp_call(h, W2, b2, dinv)
    g = h2s
    for _ in range(K_ITERS):
        acc_p = sc_prop(g, src_l, dst_l, zeros_d)
        g, h = _combine_call(acc_p, h2, h2s, dinv, dinv2)
    return h
